# baseline (speedup 1.0000x reference)
import functools
import jax
import jax.numpy as jnp
from jax import lax
from jax.experimental import pallas as pl
from jax.experimental.pallas import tpu as pltpu
from jax.experimental.pallas import tpu_sc as plsc

D = 64
SCALE = 8.0
L = 16

_info = plsc.get_sparse_core_info()
NC, NS = _info.num_cores, _info.num_subcores
NW = NC * NS

NBUF = 2


def _make_emb(R, S):
    assert R % (NW * NBUF) == 0
    rows_w = R // NW
    mesh = plsc.VectorSubcoreMesh(core_axis_name="c", subcore_axis_name="s")

    @functools.partial(
        pl.kernel, mesh=mesh,
        out_type=jax.ShapeDtypeStruct((R, S, D), jnp.float32),
        compiler_params=pltpu.CompilerParams(use_tc_tiling_on_sc=False),
        scratch_types=[
            pltpu.VMEM((rows_w, S), jnp.int32),
            pltpu.VMEM((NBUF, S, D), jnp.float32),
            pltpu.SemaphoreType.DMA,
            pltpu.SemaphoreType.DMA,
            pltpu.SemaphoreType.DMA,
            pltpu.SemaphoreType.DMA,
        ],
    )
    def _emb(x_hbm, table_hbm, out_hbm, idx_v, rows_v, g0, g1, o0, o1):
        gsem = [g0, g1]
        osem = [o0, o1]
        wid = lax.axis_index("s") * NC + lax.axis_index("c")
        base = wid * rows_w
        pltpu.sync_copy(x_hbm.at[pl.ds(base, rows_w)], idx_v)

        def g_desc(r, b):
            return pltpu.make_async_copy(
                table_hbm.at[idx_v.at[r]], rows_v.at[b], gsem[b])

        def o_desc(r, b):
            return pltpu.make_async_copy(
                rows_v.at[b], out_hbm.at[base + r], osem[b])

        def scale(b):
            def row(r, _):
                for c in range(D // L):
                    rows_v[b, r, pl.ds(c * L, L)] = (
                        rows_v[b, r, pl.ds(c * L, L)] * SCALE)
                return 0
            lax.fori_loop(0, S, row, 0)

        g_desc(0, 0).start()

        def outer(o, _):
            for b in range(NBUF):
                r = o * NBUF + b
                nb = (b + 1) % NBUF
                @pl.when(r + 1 < rows_w)
                def _():
                    @pl.when(r >= 1)
                    def _():
                        o_desc(r - 1, nb).wait()
                    g_desc(r + 1, nb).start()

                g_desc(r, b).wait()
                scale(b)
                o_desc(r, b).start()
            return 0

        lax.fori_loop(0, rows_w // NBUF, outer, 0)
        o_desc(rows_w - 2, (rows_w - 2) % NBUF).wait()
        o_desc(rows_w - 1, (rows_w - 1) % NBUF).wait()

    return _emb


def kernel(x, table):
    R, S = x.shape
    return _make_emb(R, S)(x.astype(jnp.int32), table)
